# baseline (speedup 1.0000x reference)
import functools

import jax
import jax.numpy as jnp
from jax import lax
from jax.experimental import pallas as pl
from jax.experimental.pallas import tpu as pltpu
from jax.experimental.pallas import tpu_sc as plsc

N = 10000
D = 128
E = 320000
EPS = 1e-5

NW = 32
K = 128
C = 80
G = 16
CS0 = 32
CS1 = 128
EPAD = NW * C * K
NPAD = 10240
RPT = NPAD // 16

_mesh = plsc.VectorSubcoreMesh(core_axis_name="c", subcore_axis_name="s")


@functools.partial(
    pl.kernel, mesh=_mesh,
    out_type=[jax.ShapeDtypeStruct((2, NPAD, D), jnp.float32)],
    scratch_types=[
        pltpu.VMEM_SHARED((NPAD, D), jnp.float32),
        pltpu.VMEM((G, K), jnp.int32),
        pltpu.VMEM((G, K), jnp.int32),
        pltpu.VMEM((K, D), jnp.float32),
        pltpu.VMEM((K, D), jnp.float32),
        pltpu.SemaphoreType.DMA,
        pltpu.SemaphoreType.DMA,
        pltpu.SemaphoreType.DMA,
        pltpu.SemaphoreType.DMA,
    ],
)
def _sc_agg(h_hbm, srcb, dstb, zf, part, agg_sh, src_v, dst_v, b0, b1,
            semg0, semg1, sems0, sems1):
    c = lax.axis_index("c")
    s = lax.axis_index("s")
    wid = c * 16 + s
    rb = s * RPT

    pltpu.sync_copy(zf, b0)
    for m in range(RPT // K):
        pltpu.sync_copy(b0, agg_sh.at[pl.ds(rb + m * K, K)])
    plsc.subcore_barrier()

    def group_at(base, g, carry):
        pltpu.sync_copy(srcb.at[pl.ds(base + g * G, G)], src_v)
        pltpu.sync_copy(dstb.at[pl.ds(base + g * G, G)], dst_v)
        pltpu.async_copy(h_hbm.at[src_v.at[0]], b0, semg0)

        def pair(p, carry2):
            j0 = 2 * p
            j1 = 2 * p + 1

            @pl.when(p > 0)
            def _():
                pltpu.make_async_copy(
                    b1, agg_sh.at[dst_v.at[j0 - 1]], sems1).wait()

            pltpu.async_copy(h_hbm.at[src_v.at[j1]], b1, semg1)
            pltpu.make_async_copy(h_hbm.at[src_v.at[j0]], b0, semg0).wait()
            pltpu.async_copy(b0, agg_sh.at[dst_v.at[j0]], sems0, add=True)

            @pl.when(j1 + 1 < G)
            def _():
                pltpu.make_async_copy(
                    b0, agg_sh.at[dst_v.at[j0]], sems0).wait()
                pltpu.async_copy(h_hbm.at[src_v.at[j1 + 1]], b0, semg0)

            pltpu.make_async_copy(h_hbm.at[src_v.at[j1]], b1, semg1).wait()
            pltpu.async_copy(b1, agg_sh.at[dst_v.at[j1]], sems1, add=True)
            return carry2

        lax.fori_loop(0, G // 2, pair, carry)
        pltpu.make_async_copy(b0, agg_sh.at[dst_v.at[G - 2]], sems0).wait()
        pltpu.make_async_copy(b1, agg_sh.at[dst_v.at[G - 1]], sems1).wait()
        return carry

    @pl.when(c == 0)
    def _():
        lax.fori_loop(0, CS0 // G,
                      functools.partial(group_at, s * CS0), 0)

    @pl.when(c == 1)
    def _():
        lax.fori_loop(0, CS1 // G,
                      functools.partial(group_at, 16 * CS0 + s * CS1), 0)
    plsc.subcore_barrier()

    for m in range(RPT // K):
        pltpu.sync_copy(agg_sh.at[pl.ds(rb + m * K, K)], b0)
        pltpu.sync_copy(b0, part.at[c, pl.ds(rb + m * K, K)])


@functools.partial(
    pl.kernel, mesh=_mesh,
    out_type=[jax.ShapeDtypeStruct((2, NPAD, D), jnp.float32)],
    scratch_types=[
        pltpu.VMEM_SHARED((NPAD, D), jnp.float32),
        pltpu.VMEM((G, K), jnp.int32),
        pltpu.VMEM((K, D), jnp.float32),
        pltpu.SemaphoreType.DMA,
    ],
)
def _sc_cnt(dstb, zf, of, cntp, cnt_sh, dst_v, buf, sem):
    c = lax.axis_index("c")
    s = lax.axis_index("s")
    wid = c * 16 + s
    rb = s * RPT

    pltpu.sync_copy(zf, buf)
    for m in range(RPT // K):
        pltpu.sync_copy(buf, cnt_sh.at[pl.ds(rb + m * K, K)])
    plsc.subcore_barrier()
    pltpu.sync_copy(of, buf)

    def group(g, carry):
        pltpu.sync_copy(dstb.at[pl.ds(wid * C + g * G, G)], dst_v)

        def chunk(j, carry2):
            pltpu.sync_copy(buf, cnt_sh.at[dst_v.at[j]], add=True)
            return carry2

        return lax.fori_loop(0, G, chunk, carry)

    lax.fori_loop(0, C // G, group, 0)
    plsc.subcore_barrier()

    for m in range(RPT // K):
        pltpu.sync_copy(cnt_sh.at[pl.ds(rb + m * K, K)], buf)
        pltpu.sync_copy(buf, cntp.at[c, pl.ds(rb + m * K, K)])


BR = 1000


def _tc_a_body(part, cntp, h, wlt, wrt, bl, t_out, stats_out, acc):
    i = pl.program_id(0)
    cnt = cntp[0, :, 0:1] + cntp[1, :, 0:1]
    agg = (part[0] + part[1]) / jnp.maximum(cnt, 1.0)
    t = (jnp.dot(agg, wlt[...], preferred_element_type=jnp.float32)
         + jnp.dot(h[...], wrt[...], preferred_element_type=jnp.float32)
         + bl[...])
    t_out[...] = t

    @pl.when(i == 0)
    def _():
        acc[...] = jnp.zeros_like(acc)

    acc[0:1, :] += jnp.sum(t, axis=0, keepdims=True)
    acc[1:2, :] += jnp.sum(t * t, axis=0, keepdims=True)

    @pl.when(i == (N // BR) - 1)
    def _():
        stats_out[...] = acc[...]


_tc_a = pl.pallas_call(
    _tc_a_body,
    grid=(N // BR,),
    in_specs=[
        pl.BlockSpec((2, BR, D), lambda i: (0, i, 0)),
        pl.BlockSpec((2, BR, D), lambda i: (0, i, 0)),
        pl.BlockSpec((BR, D), lambda i: (i, 0)),
        pl.BlockSpec((D, D), lambda i: (0, 0)),
        pl.BlockSpec((D, D), lambda i: (0, 0)),
        pl.BlockSpec((1, D), lambda i: (0, 0)),
    ],
    out_specs=[
        pl.BlockSpec((BR, D), lambda i: (i, 0)),
        pl.BlockSpec((8, D), lambda i: (0, 0)),
    ],
    out_shape=[
        jax.ShapeDtypeStruct((N, D), jnp.float32),
        jax.ShapeDtypeStruct((8, D), jnp.float32),
    ],
    scratch_shapes=[pltpu.VMEM((8, D), jnp.float32)],
)


def _tc_b_body(t, stats, gamma, beta, out, *, relu):
    mu = stats[0:1, :] * (1.0 / N)
    var = stats[1:2, :] * (1.0 / N) - mu * mu
    scale = gamma[...] * lax.rsqrt(var + EPS)
    y = (t[...] - mu) * scale + beta[...]
    if relu:
        y = jnp.maximum(y, 0.0)
    out[...] = y


_tc_b_relu = pl.pallas_call(
    functools.partial(_tc_b_body, relu=True),
    grid=(N // BR,),
    in_specs=[
        pl.BlockSpec((BR, D), lambda i: (i, 0)),
        pl.BlockSpec((8, D), lambda i: (0, 0)),
        pl.BlockSpec((1, D), lambda i: (0, 0)),
        pl.BlockSpec((1, D), lambda i: (0, 0)),
    ],
    out_specs=pl.BlockSpec((BR, D), lambda i: (i, 0)),
    out_shape=jax.ShapeDtypeStruct((N, D), jnp.float32),
)


def _tc_b_final_body(t, stats, gamma, beta, wft, bf, out):
    mu = stats[0:1, :] * (1.0 / N)
    var = stats[1:2, :] * (1.0 / N) - mu * mu
    scale = gamma[...] * lax.rsqrt(var + EPS)
    y = (t[...] - mu) * scale + beta[...]
    out[...] = jnp.dot(y, wft[...], preferred_element_type=jnp.float32) + bf[...]


_tc_b_final = pl.pallas_call(
    _tc_b_final_body,
    grid=(N // BR,),
    in_specs=[
        pl.BlockSpec((BR, D), lambda i: (i, 0)),
        pl.BlockSpec((8, D), lambda i: (0, 0)),
        pl.BlockSpec((1, D), lambda i: (0, 0)),
        pl.BlockSpec((1, D), lambda i: (0, 0)),
        pl.BlockSpec((D, D), lambda i: (0, 0)),
        pl.BlockSpec((1, D), lambda i: (0, 0)),
    ],
    out_specs=pl.BlockSpec((BR, D), lambda i: (i, 0)),
    out_shape=jax.ShapeDtypeStruct((N, D), jnp.float32),
)


def kernel(x, edge_index, params):
    src = edge_index[0]
    dst = edge_index[1]
    pad = EPAD - E
    srcb = jnp.concatenate([src, jnp.zeros((pad,), jnp.int32)]).reshape(NW * C, K)
    dstb = jnp.concatenate([dst, jnp.full((pad,), N, jnp.int32)]).reshape(NW * C, K)

    zf = jnp.zeros((K, D), jnp.float32)
    of = jnp.ones((K, D), jnp.float32)

    (cntp,) = _sc_cnt(dstb, zf, of)

    h = x
    for i in range(4):
        (part,) = _sc_agg(h, srcb, dstb, zf)
        wlt = params["Wl"][i].T
        wrt = params["Wr"][i].T
        bl = params["bl"][i].reshape(1, D)
        t, stats = _tc_a(part, cntp, h, wlt, wrt, bl)
        gamma = params["gamma"][i].reshape(1, D)
        beta = params["beta"][i].reshape(1, D)
        if i < 3:
            h = _tc_b_relu(t, stats, gamma, beta)
        else:
            h = _tc_b_final(t, stats, gamma, beta, params["Wf"].T,
                            params["bf"].reshape(1, D))
    return h


# speedup vs baseline: 1.1758x; 1.1758x over previous
import functools

import jax
import jax.numpy as jnp
from jax import lax
from jax.experimental import pallas as pl
from jax.experimental.pallas import tpu as pltpu
from jax.experimental.pallas import tpu_sc as plsc

N = 10000
D = 128
E = 320000
EPS = 1e-5

NW = 32
K = 128
C = 80
G = 16
CS0 = 128
CS1 = 32
EPAD = NW * C * K
NPAD = 10240
RPT = NPAD // 16

_mesh = plsc.VectorSubcoreMesh(core_axis_name="c", subcore_axis_name="s")


@functools.partial(
    pl.kernel, mesh=_mesh,
    out_type=[jax.ShapeDtypeStruct((2, NPAD, D), jnp.float32)],
    scratch_types=[
        pltpu.VMEM_SHARED((NPAD, D), jnp.float32),
        pltpu.VMEM((G, K), jnp.int32),
        pltpu.VMEM((G, K), jnp.int32),
        pltpu.VMEM((K, D), jnp.float32),
        pltpu.VMEM((K, D), jnp.float32),
        pltpu.SemaphoreType.DMA,
        pltpu.SemaphoreType.DMA,
        pltpu.SemaphoreType.DMA,
        pltpu.SemaphoreType.DMA,
    ],
)
def _sc_agg(h_hbm, srcb, dstb, zf, part, agg_sh, src_v, dst_v, b0, b1,
            semg0, semg1, sems0, sems1):
    c = lax.axis_index("c")
    s = lax.axis_index("s")
    wid = c * 16 + s
    rb = s * RPT

    pltpu.sync_copy(zf, b0)
    for m in range(RPT // K):
        pltpu.sync_copy(b0, agg_sh.at[pl.ds(rb + m * K, K)])
    plsc.subcore_barrier()

    def group_at(base, g, carry):
        pltpu.sync_copy(srcb.at[pl.ds(base + g * G, G)], src_v)
        pltpu.sync_copy(dstb.at[pl.ds(base + g * G, G)], dst_v)
        pltpu.async_copy(h_hbm.at[src_v.at[0]], b0, semg0)

        def pair(p, carry2):
            j0 = 2 * p
            j1 = 2 * p + 1

            @pl.when(p > 0)
            def _():
                pltpu.make_async_copy(
                    b1, agg_sh.at[dst_v.at[j0 - 1]], sems1).wait()

            pltpu.async_copy(h_hbm.at[src_v.at[j1]], b1, semg1)
            pltpu.make_async_copy(h_hbm.at[src_v.at[j0]], b0, semg0).wait()
            pltpu.async_copy(b0, agg_sh.at[dst_v.at[j0]], sems0, add=True)

            @pl.when(j1 + 1 < G)
            def _():
                pltpu.make_async_copy(
                    b0, agg_sh.at[dst_v.at[j0]], sems0).wait()
                pltpu.async_copy(h_hbm.at[src_v.at[j1 + 1]], b0, semg0)

            pltpu.make_async_copy(h_hbm.at[src_v.at[j1]], b1, semg1).wait()
            pltpu.async_copy(b1, agg_sh.at[dst_v.at[j1]], sems1, add=True)
            return carry2

        lax.fori_loop(0, G // 2, pair, carry)
        pltpu.make_async_copy(b0, agg_sh.at[dst_v.at[G - 2]], sems0).wait()
        pltpu.make_async_copy(b1, agg_sh.at[dst_v.at[G - 1]], sems1).wait()
        return carry

    @pl.when(c == 0)
    def _():
        lax.fori_loop(0, CS0 // G,
                      functools.partial(group_at, s * CS0), 0)

    @pl.when(c == 1)
    def _():
        lax.fori_loop(0, CS1 // G,
                      functools.partial(group_at, 16 * CS0 + s * CS1), 0)
    plsc.subcore_barrier()

    for m in range(RPT // K):
        pltpu.sync_copy(agg_sh.at[pl.ds(rb + m * K, K)], b0)
        pltpu.sync_copy(b0, part.at[c, pl.ds(rb + m * K, K)])


@functools.partial(
    pl.kernel, mesh=_mesh,
    out_type=[jax.ShapeDtypeStruct((2, NPAD, D), jnp.float32)],
    scratch_types=[
        pltpu.VMEM_SHARED((NPAD, D), jnp.float32),
        pltpu.VMEM((G, K), jnp.int32),
        pltpu.VMEM((K, D), jnp.float32),
        pltpu.SemaphoreType.DMA,
    ],
)
def _sc_cnt(dstb, zf, of, cntp, cnt_sh, dst_v, buf, sem):
    c = lax.axis_index("c")
    s = lax.axis_index("s")
    wid = c * 16 + s
    rb = s * RPT

    pltpu.sync_copy(zf, buf)
    for m in range(RPT // K):
        pltpu.sync_copy(buf, cnt_sh.at[pl.ds(rb + m * K, K)])
    plsc.subcore_barrier()
    pltpu.sync_copy(of, buf)

    def group(g, carry):
        pltpu.sync_copy(dstb.at[pl.ds(wid * C + g * G, G)], dst_v)

        def chunk(j, carry2):
            pltpu.sync_copy(buf, cnt_sh.at[dst_v.at[j]], add=True)
            return carry2

        return lax.fori_loop(0, G, chunk, carry)

    lax.fori_loop(0, C // G, group, 0)
    plsc.subcore_barrier()

    for m in range(RPT // K):
        pltpu.sync_copy(cnt_sh.at[pl.ds(rb + m * K, K)], buf)
        pltpu.sync_copy(buf, cntp.at[c, pl.ds(rb + m * K, K)])


BR = 1000


def _tc_a_body(part, cntp, h, wlt, wrt, bl, t_out, stats_out, acc):
    i = pl.program_id(0)
    cnt = cntp[0, :, 0:1] + cntp[1, :, 0:1]
    agg = (part[0] + part[1]) / jnp.maximum(cnt, 1.0)
    t = (jnp.dot(agg, wlt[...], preferred_element_type=jnp.float32)
         + jnp.dot(h[...], wrt[...], preferred_element_type=jnp.float32)
         + bl[...])
    t_out[...] = t

    @pl.when(i == 0)
    def _():
        acc[...] = jnp.zeros_like(acc)

    acc[0:1, :] += jnp.sum(t, axis=0, keepdims=True)
    acc[1:2, :] += jnp.sum(t * t, axis=0, keepdims=True)

    @pl.when(i == (N // BR) - 1)
    def _():
        stats_out[...] = acc[...]


_tc_a = pl.pallas_call(
    _tc_a_body,
    grid=(N // BR,),
    in_specs=[
        pl.BlockSpec((2, BR, D), lambda i: (0, i, 0)),
        pl.BlockSpec((2, BR, D), lambda i: (0, i, 0)),
        pl.BlockSpec((BR, D), lambda i: (i, 0)),
        pl.BlockSpec((D, D), lambda i: (0, 0)),
        pl.BlockSpec((D, D), lambda i: (0, 0)),
        pl.BlockSpec((1, D), lambda i: (0, 0)),
    ],
    out_specs=[
        pl.BlockSpec((BR, D), lambda i: (i, 0)),
        pl.BlockSpec((8, D), lambda i: (0, 0)),
    ],
    out_shape=[
        jax.ShapeDtypeStruct((N, D), jnp.float32),
        jax.ShapeDtypeStruct((8, D), jnp.float32),
    ],
    scratch_shapes=[pltpu.VMEM((8, D), jnp.float32)],
)


def _tc_b_body(t, stats, gamma, beta, out, *, relu):
    mu = stats[0:1, :] * (1.0 / N)
    var = stats[1:2, :] * (1.0 / N) - mu * mu
    scale = gamma[...] * lax.rsqrt(var + EPS)
    y = (t[...] - mu) * scale + beta[...]
    if relu:
        y = jnp.maximum(y, 0.0)
    out[...] = y


_tc_b_relu = pl.pallas_call(
    functools.partial(_tc_b_body, relu=True),
    grid=(N // BR,),
    in_specs=[
        pl.BlockSpec((BR, D), lambda i: (i, 0)),
        pl.BlockSpec((8, D), lambda i: (0, 0)),
        pl.BlockSpec((1, D), lambda i: (0, 0)),
        pl.BlockSpec((1, D), lambda i: (0, 0)),
    ],
    out_specs=pl.BlockSpec((BR, D), lambda i: (i, 0)),
    out_shape=jax.ShapeDtypeStruct((N, D), jnp.float32),
)


def _tc_b_final_body(t, stats, gamma, beta, wft, bf, out):
    mu = stats[0:1, :] * (1.0 / N)
    var = stats[1:2, :] * (1.0 / N) - mu * mu
    scale = gamma[...] * lax.rsqrt(var + EPS)
    y = (t[...] - mu) * scale + beta[...]
    out[...] = jnp.dot(y, wft[...], preferred_element_type=jnp.float32) + bf[...]


_tc_b_final = pl.pallas_call(
    _tc_b_final_body,
    grid=(N // BR,),
    in_specs=[
        pl.BlockSpec((BR, D), lambda i: (i, 0)),
        pl.BlockSpec((8, D), lambda i: (0, 0)),
        pl.BlockSpec((1, D), lambda i: (0, 0)),
        pl.BlockSpec((1, D), lambda i: (0, 0)),
        pl.BlockSpec((D, D), lambda i: (0, 0)),
        pl.BlockSpec((1, D), lambda i: (0, 0)),
    ],
    out_specs=pl.BlockSpec((BR, D), lambda i: (i, 0)),
    out_shape=jax.ShapeDtypeStruct((N, D), jnp.float32),
)


def kernel(x, edge_index, params):
    src = edge_index[0]
    dst = edge_index[1]
    pad = EPAD - E
    srcb = jnp.concatenate([src, jnp.zeros((pad,), jnp.int32)]).reshape(NW * C, K)
    dstb = jnp.concatenate([dst, jnp.full((pad,), N, jnp.int32)]).reshape(NW * C, K)

    zf = jnp.zeros((K, D), jnp.float32)
    of = jnp.ones((K, D), jnp.float32)

    (cntp,) = _sc_cnt(dstb, zf, of)

    h = x
    for i in range(4):
        (part,) = _sc_agg(h, srcb, dstb, zf)
        wlt = params["Wl"][i].T
        wrt = params["Wr"][i].T
        bl = params["bl"][i].reshape(1, D)
        t, stats = _tc_a(part, cntp, h, wlt, wrt, bl)
        gamma = params["gamma"][i].reshape(1, D)
        beta = params["beta"][i].reshape(1, D)
        if i < 3:
            h = _tc_b_relu(t, stats, gamma, beta)
        else:
            h = _tc_b_final(t, stats, gamma, beta, params["Wf"].T,
                            params["bf"].reshape(1, D))
    return h


# speedup vs baseline: 1.3310x; 1.1320x over previous
import functools

import jax
import jax.numpy as jnp
from jax import lax
from jax.experimental import pallas as pl
from jax.experimental.pallas import tpu as pltpu
from jax.experimental.pallas import tpu_sc as plsc

N = 10000
D = 128
E = 320000
EPS = 1e-5

NW = 32
K = 128
C = 80
G = 16
CS0 = 144
CS1 = 16
EPAD = NW * C * K
NPAD = 10240
RPT = NPAD // 16

_mesh = plsc.VectorSubcoreMesh(core_axis_name="c", subcore_axis_name="s")


@functools.partial(
    pl.kernel, mesh=_mesh,
    out_type=[jax.ShapeDtypeStruct((2, NPAD, D), jnp.float32)],
    scratch_types=[
        pltpu.VMEM_SHARED((NPAD, D), jnp.float32),
        pltpu.VMEM((G, K), jnp.int32),
        pltpu.VMEM((G, K), jnp.int32),
        pltpu.VMEM((K, D), jnp.float32),
        pltpu.VMEM((K, D), jnp.float32),
        pltpu.SemaphoreType.DMA,
        pltpu.SemaphoreType.DMA,
        pltpu.SemaphoreType.DMA,
        pltpu.SemaphoreType.DMA,
    ],
)
def _sc_agg(h_hbm, srcb, dstb, zf, part, agg_sh, src_v, dst_v, b0, b1,
            semg0, semg1, sems0, sems1):
    c = lax.axis_index("c")
    s = lax.axis_index("s")
    wid = c * 16 + s
    rb = s * RPT

    pltpu.sync_copy(zf, b0)
    for m in range(RPT // K):
        pltpu.sync_copy(b0, agg_sh.at[pl.ds(rb + m * K, K)])
    plsc.subcore_barrier()

    def group_at(base, g, carry):
        pltpu.sync_copy(srcb.at[pl.ds(base + g * G, G)], src_v)
        pltpu.sync_copy(dstb.at[pl.ds(base + g * G, G)], dst_v)
        pltpu.async_copy(h_hbm.at[src_v.at[0]], b0, semg0)

        def pair(p, carry2):
            j0 = 2 * p
            j1 = 2 * p + 1

            @pl.when(p > 0)
            def _():
                pltpu.make_async_copy(
                    b1, agg_sh.at[dst_v.at[j0 - 1]], sems1).wait()

            pltpu.async_copy(h_hbm.at[src_v.at[j1]], b1, semg1)
            pltpu.make_async_copy(h_hbm.at[src_v.at[j0]], b0, semg0).wait()
            pltpu.async_copy(b0, agg_sh.at[dst_v.at[j0]], sems0, add=True)

            @pl.when(j1 + 1 < G)
            def _():
                pltpu.make_async_copy(
                    b0, agg_sh.at[dst_v.at[j0]], sems0).wait()
                pltpu.async_copy(h_hbm.at[src_v.at[j1 + 1]], b0, semg0)

            pltpu.make_async_copy(h_hbm.at[src_v.at[j1]], b1, semg1).wait()
            pltpu.async_copy(b1, agg_sh.at[dst_v.at[j1]], sems1, add=True)
            return carry2

        lax.fori_loop(0, G // 2, pair, carry)
        pltpu.make_async_copy(b0, agg_sh.at[dst_v.at[G - 2]], sems0).wait()
        pltpu.make_async_copy(b1, agg_sh.at[dst_v.at[G - 1]], sems1).wait()
        return carry

    @pl.when(c == 0)
    def _():
        lax.fori_loop(0, CS0 // G,
                      functools.partial(group_at, s * CS0), 0)

    @pl.when(c == 1)
    def _():
        lax.fori_loop(0, CS1 // G,
                      functools.partial(group_at, 16 * CS0 + s * CS1), 0)
    plsc.subcore_barrier()

    for m in range(RPT // K):
        pltpu.sync_copy(agg_sh.at[pl.ds(rb + m * K, K)], b0)
        pltpu.sync_copy(b0, part.at[c, pl.ds(rb + m * K, K)])


@functools.partial(
    pl.kernel, mesh=_mesh,
    out_type=[jax.ShapeDtypeStruct((2, NPAD, D), jnp.float32)],
    scratch_types=[
        pltpu.VMEM_SHARED((NPAD, D), jnp.float32),
        pltpu.VMEM((G, K), jnp.int32),
        pltpu.VMEM((K, D), jnp.float32),
        pltpu.SemaphoreType.DMA,
    ],
)
def _sc_cnt(dstb, zf, of, cntp, cnt_sh, dst_v, buf, sem):
    c = lax.axis_index("c")
    s = lax.axis_index("s")
    wid = c * 16 + s
    rb = s * RPT

    pltpu.sync_copy(zf, buf)
    for m in range(RPT // K):
        pltpu.sync_copy(buf, cnt_sh.at[pl.ds(rb + m * K, K)])
    plsc.subcore_barrier()
    pltpu.sync_copy(of, buf)

    def group(g, carry):
        pltpu.sync_copy(dstb.at[pl.ds(wid * C + g * G, G)], dst_v)

        def chunk(j, carry2):
            pltpu.sync_copy(buf, cnt_sh.at[dst_v.at[j]], add=True)
            return carry2

        return lax.fori_loop(0, G, chunk, carry)

    lax.fori_loop(0, C // G, group, 0)
    plsc.subcore_barrier()

    for m in range(RPT // K):
        pltpu.sync_copy(cnt_sh.at[pl.ds(rb + m * K, K)], buf)
        pltpu.sync_copy(buf, cntp.at[c, pl.ds(rb + m * K, K)])


BR = 1000


def _tc_a_body(part, cntp, h, wlt, wrt, bl, t_out, stats_out, acc):
    i = pl.program_id(0)
    cnt = cntp[0, :, 0:1] + cntp[1, :, 0:1]
    agg = (part[0] + part[1]) / jnp.maximum(cnt, 1.0)
    t = (jnp.dot(agg, wlt[...], preferred_element_type=jnp.float32)
         + jnp.dot(h[...], wrt[...], preferred_element_type=jnp.float32)
         + bl[...])
    t_out[...] = t

    @pl.when(i == 0)
    def _():
        acc[...] = jnp.zeros_like(acc)

    acc[0:1, :] += jnp.sum(t, axis=0, keepdims=True)
    acc[1:2, :] += jnp.sum(t * t, axis=0, keepdims=True)

    @pl.when(i == (N // BR) - 1)
    def _():
        stats_out[...] = acc[...]


_tc_a = pl.pallas_call(
    _tc_a_body,
    grid=(N // BR,),
    in_specs=[
        pl.BlockSpec((2, BR, D), lambda i: (0, i, 0)),
        pl.BlockSpec((2, BR, D), lambda i: (0, i, 0)),
        pl.BlockSpec((BR, D), lambda i: (i, 0)),
        pl.BlockSpec((D, D), lambda i: (0, 0)),
        pl.BlockSpec((D, D), lambda i: (0, 0)),
        pl.BlockSpec((1, D), lambda i: (0, 0)),
    ],
    out_specs=[
        pl.BlockSpec((BR, D), lambda i: (i, 0)),
        pl.BlockSpec((8, D), lambda i: (0, 0)),
    ],
    out_shape=[
        jax.ShapeDtypeStruct((N, D), jnp.float32),
        jax.ShapeDtypeStruct((8, D), jnp.float32),
    ],
    scratch_shapes=[pltpu.VMEM((8, D), jnp.float32)],
)


def _tc_b_body(t, stats, gamma, beta, out, *, relu):
    mu = stats[0:1, :] * (1.0 / N)
    var = stats[1:2, :] * (1.0 / N) - mu * mu
    scale = gamma[...] * lax.rsqrt(var + EPS)
    y = (t[...] - mu) * scale + beta[...]
    if relu:
        y = jnp.maximum(y, 0.0)
    out[...] = y


_tc_b_relu = pl.pallas_call(
    functools.partial(_tc_b_body, relu=True),
    grid=(N // BR,),
    in_specs=[
        pl.BlockSpec((BR, D), lambda i: (i, 0)),
        pl.BlockSpec((8, D), lambda i: (0, 0)),
        pl.BlockSpec((1, D), lambda i: (0, 0)),
        pl.BlockSpec((1, D), lambda i: (0, 0)),
    ],
    out_specs=pl.BlockSpec((BR, D), lambda i: (i, 0)),
    out_shape=jax.ShapeDtypeStruct((N, D), jnp.float32),
)


def _tc_b_final_body(t, stats, gamma, beta, wft, bf, out):
    mu = stats[0:1, :] * (1.0 / N)
    var = stats[1:2, :] * (1.0 / N) - mu * mu
    scale = gamma[...] * lax.rsqrt(var + EPS)
    y = (t[...] - mu) * scale + beta[...]
    out[...] = jnp.dot(y, wft[...], preferred_element_type=jnp.float32) + bf[...]


_tc_b_final = pl.pallas_call(
    _tc_b_final_body,
    grid=(N // BR,),
    in_specs=[
        pl.BlockSpec((BR, D), lambda i: (i, 0)),
        pl.BlockSpec((8, D), lambda i: (0, 0)),
        pl.BlockSpec((1, D), lambda i: (0, 0)),
        pl.BlockSpec((1, D), lambda i: (0, 0)),
        pl.BlockSpec((D, D), lambda i: (0, 0)),
        pl.BlockSpec((1, D), lambda i: (0, 0)),
    ],
    out_specs=pl.BlockSpec((BR, D), lambda i: (i, 0)),
    out_shape=jax.ShapeDtypeStruct((N, D), jnp.float32),
)


def kernel(x, edge_index, params):
    src = edge_index[0]
    dst = edge_index[1]
    pad = EPAD - E
    srcb = jnp.concatenate([src, jnp.zeros((pad,), jnp.int32)]).reshape(NW * C, K)
    dstb = jnp.concatenate([dst, jnp.full((pad,), N, jnp.int32)]).reshape(NW * C, K)

    zf = jnp.zeros((K, D), jnp.float32)
    of = jnp.ones((K, D), jnp.float32)

    (cntp,) = _sc_cnt(dstb, zf, of)

    h = x
    for i in range(4):
        (part,) = _sc_agg(h, srcb, dstb, zf)
        wlt = params["Wl"][i].T
        wrt = params["Wr"][i].T
        bl = params["bl"][i].reshape(1, D)
        t, stats = _tc_a(part, cntp, h, wlt, wrt, bl)
        gamma = params["gamma"][i].reshape(1, D)
        beta = params["beta"][i].reshape(1, D)
        if i < 3:
            h = _tc_b_relu(t, stats, gamma, beta)
        else:
            h = _tc_b_final(t, stats, gamma, beta, params["Wf"].T,
                            params["bf"].reshape(1, D))
    return h
